# baseline (speedup 1.0000x reference)
import functools

import jax
import jax.numpy as jnp
from jax import lax
from jax.experimental import pallas as pl
from jax.experimental.pallas import tpu as pltpu
from jax.experimental.pallas import tpu_sc as plsc

KNN = 3
BATCH_OFF = 1000.0
_BIG = 2**30


def _insert3(state, cand):
    out = []
    e = cand
    for s in state:
        t = e[0] < s[0]
        keep = tuple(jnp.where(t, a, b) for a, b in zip(e, s))
        e = tuple(jnp.where(t, b, a) for a, b in zip(e, s))
        out.append(keep)
    return tuple(out)


def _knn_body(lo_ref, nt_ref, yc_ref, ysq_ref, xs_ref, xsq_ref, idx_ref,
              *, block_rows, col_tile):
    blk = pl.program_id(0)
    lo = lo_ref[blk]
    ntiles = nt_ref[blk]
    yc = yc_ref[...]
    ysq = ysq_ref[...]

    r = block_rows
    inf = jnp.float32(jnp.inf)
    siota = jax.lax.broadcasted_iota(jnp.int32, (col_tile, r), 0)

    def tile_step(t, state):
        start = pl.multiple_of(lo + t * col_tile, 128)
        xsT = xs_ref[:, pl.ds(start, col_tile)]
        xsq = xsq_ref[:, pl.ds(start, col_tile)]
        dot = jax.lax.dot_general(yc, xsT, (((1,), (0,)), ((), ())),
                                  preferred_element_type=jnp.float32)
        d2 = jnp.transpose((ysq + xsq) - 2.0 * dot)
        for k in range(KNN):
            m = jnp.min(d2, axis=0, keepdims=True)
            cand = jnp.where(d2 == m, siota, jnp.int32(_BIG))
            il = jnp.min(cand, axis=0, keepdims=True)
            state = _insert3(state, (m, il + start))
            if k < KNN - 1:
                d2 = jnp.where(siota == il, inf, d2)
        return state

    def entry():
        return (jnp.full((1, r), inf), jnp.full((1, r), _BIG, jnp.int32))

    state = jax.lax.fori_loop(0, ntiles, tile_step,
                              (entry(), entry(), entry()))
    for k in range(KNN):
        idx_ref[k, :] = state[k][1][0]


def _knn_topk(off_y, ysq, xsT_pad, xsq_pad, lo_arr, nt_arr,
              block_rows=256, col_tile=512):
    ny = off_y.shape[0]
    nxp = xsT_pad.shape[1]
    grid = (ny // block_rows,)
    body = functools.partial(_knn_body, block_rows=block_rows,
                             col_tile=col_tile)
    return pl.pallas_call(
        body,
        grid_spec=pltpu.PrefetchScalarGridSpec(
            num_scalar_prefetch=2,
            grid=grid,
            in_specs=[
                pl.BlockSpec((block_rows, 3), lambda i, lo, nt: (i, 0)),
                pl.BlockSpec((block_rows, 1), lambda i, lo, nt: (i, 0)),
                pl.BlockSpec((3, nxp), lambda i, lo, nt: (0, 0)),
                pl.BlockSpec((1, nxp), lambda i, lo, nt: (0, 0)),
            ],
            out_specs=pl.BlockSpec((KNN, block_rows),
                                   lambda i, lo, nt: (0, i)),
        ),
        out_shape=jax.ShapeDtypeStruct((KNN, ny), jnp.int32),
    )(lo_arr, nt_arr, off_y, ysq, xsT_pad, xsq_pad)


def _lane_bcast(v, lane):
    idx = jnp.full((16, 1), lane, jnp.int32)
    dn = lax.GatherDimensionNumbers(offset_dims=(), collapsed_slice_dims=(0,),
                                    start_index_map=(0,))
    return lax.gather(v, idx, dn, (1,),
                      mode=lax.GatherScatterMode.PROMISE_IN_BOUNDS)


def _interp_sc(x, oxp, oyp, idx_rows, chunk=128):
    nx, f = x.shape
    ny = idx_rows[0].shape[0]
    info = plsc.get_sparse_core_info()
    nw = info.num_cores * info.num_subcores
    rw = ny // nw
    nchunks = rw // chunk
    mesh = plsc.VectorSubcoreMesh(core_axis_name="c", subcore_axis_name="s")

    def body(x_h, oxp_h, oyp_h, i0_h, i1_h, i2_h, out_h,
             iv0, iv1, iv2, p0, p1, p2, yv, g0, g1, g2, ov, sem):
        wid = lax.axis_index("s") * info.num_cores + lax.axis_index("c")

        def chunk_body(ci, carry):
            base = wid * rw + ci * chunk
            pltpu.sync_copy(i0_h.at[pl.ds(base, chunk)], iv0)
            pltpu.sync_copy(i1_h.at[pl.ds(base, chunk)], iv1)
            pltpu.sync_copy(i2_h.at[pl.ds(base, chunk)], iv2)
            c0 = pltpu.async_copy(x_h.at[iv0], g0, sem)
            c1 = pltpu.async_copy(x_h.at[iv1], g1, sem)
            c2 = pltpu.async_copy(x_h.at[iv2], g2, sem)
            c3 = pltpu.async_copy(oxp_h.at[iv0], p0, sem)
            c4 = pltpu.async_copy(oxp_h.at[iv1], p1, sem)
            c5 = pltpu.async_copy(oxp_h.at[iv2], p2, sem)
            pltpu.sync_copy(oyp_h.at[pl.ds(base, chunk)], yv)
            for c in (c0, c1, c2, c3, c4, c5):
                c.wait()

            def one_row(r):
                yrow = yv[r, :]

                def d2sel(pref):
                    dd = yrow - pref[r, :]
                    s = dd * dd
                    d2s = (_lane_bcast(s, 0) + _lane_bcast(s, 1)) + _lane_bcast(s, 2)
                    return jnp.maximum(d2s, 1e-16)

                d0 = d2sel(p0)
                d1 = d2sel(p1)
                d2_ = d2sel(p2)
                p01 = d0 * d1
                p02 = d0 * d2_
                p12 = d1 * d2_
                inv = 1.0 / ((p12 + p02) + p01)
                a0 = p12 * inv
                a1 = p02 * inv
                a2 = p01 * inv
                for fi in range(f // 16):
                    fs = pl.ds(fi * 16, 16)
                    ov[r, fs] = (a0 * g0[r, fs] + a1 * g1[r, fs]) + a2 * g2[r, fs]

            unroll = 4

            def row_body(ri, c):
                for u in range(unroll):
                    one_row(ri * unroll + u)
                return c

            lax.fori_loop(0, chunk // unroll, row_body, 0)
            pltpu.sync_copy(ov, out_h.at[pl.ds(base, chunk)])
            return carry

        lax.fori_loop(0, nchunks, chunk_body, 0)

    return pl.kernel(
        body,
        out_type=jax.ShapeDtypeStruct((ny, f), jnp.float32),
        mesh=mesh,
        compiler_params=pltpu.CompilerParams(use_tc_tiling_on_sc=False),
        scratch_types=[
            pltpu.VMEM((chunk,), jnp.int32),
            pltpu.VMEM((chunk,), jnp.int32),
            pltpu.VMEM((chunk,), jnp.int32),
            pltpu.VMEM((chunk, 16), jnp.float32),
            pltpu.VMEM((chunk, 16), jnp.float32),
            pltpu.VMEM((chunk, 16), jnp.float32),
            pltpu.VMEM((chunk, 16), jnp.float32),
            pltpu.VMEM((chunk, f), jnp.float32),
            pltpu.VMEM((chunk, f), jnp.float32),
            pltpu.VMEM((chunk, f), jnp.float32),
            pltpu.VMEM((chunk, f), jnp.float32),
            pltpu.SemaphoreType.DMA,
        ],
    )(x, oxp, oyp, idx_rows[0], idx_rows[1], idx_rows[2])


def _lin_body(xi_ref, xs_ref, w1_ref, w2_ref, b_ref, o_ref):
    acc = jax.lax.dot_general(xi_ref[...], w1_ref[...],
                              (((1,), (0,)), ((), ())),
                              preferred_element_type=jnp.float32)
    acc = acc + jax.lax.dot_general(xs_ref[...], w2_ref[...],
                                    (((1,), (0,)), ((), ())),
                                    preferred_element_type=jnp.float32)
    o_ref[...] = acc + b_ref[...]


def _linear(xi, x_skip, W, b, block_rows=1024):
    ny, f = xi.shape
    w1 = W[:f]
    w2 = W[f:]
    b2d = b[None, :]
    grid = (ny // block_rows,)
    return pl.pallas_call(
        _lin_body,
        grid=grid,
        in_specs=[
            pl.BlockSpec((block_rows, f), lambda i: (i, 0)),
            pl.BlockSpec((block_rows, f), lambda i: (i, 0)),
            pl.BlockSpec((f, f), lambda i: (0, 0)),
            pl.BlockSpec((f, f), lambda i: (0, 0)),
            pl.BlockSpec((1, f), lambda i: (0, 0)),
        ],
        out_specs=pl.BlockSpec((block_rows, f), lambda i: (i, 0)),
        out_shape=jax.ShapeDtypeStruct((ny, f), jnp.float32),
    )(xi, x_skip, w1, w2, b2d)


def kernel(x, pos, batch, x_skip, pos_skip, batch_skip, W, b):
    nx = x.shape[0]
    ny = x_skip.shape[0]
    block_rows = 256
    col_tile = 512

    off_x = pos + BATCH_OFF * batch[:, None].astype(pos.dtype)
    off_y = pos_skip + BATCH_OFF * batch_skip[:, None].astype(pos_skip.dtype)
    xsq = jnp.sum(off_x * off_x, axis=-1)
    ysq = jnp.sum(off_y * off_y, axis=-1)

    nblk = ny // block_rows
    bs2 = batch_skip.reshape(nblk, block_rows)
    bmin = bs2[:, 0]
    bmax = bs2[:, -1]
    vals = jnp.arange(8, dtype=batch.dtype)
    starts = jnp.sum(batch[None, :] < vals[:, None], axis=1).astype(jnp.int32)
    ends = jnp.sum(batch[None, :] <= vals[:, None], axis=1).astype(jnp.int32)
    seg_lo = starts[bmin]
    seg_hi = ends[bmax]
    lo_arr = (seg_lo // 128) * 128
    nt_arr = (seg_hi - lo_arr + (col_tile - 1)) // col_tile

    xsT_pad = jnp.pad(off_x.T, ((0, 0), (0, col_tile)))
    xsq_pad = jnp.pad(xsq[None, :], ((0, 0), (0, col_tile)))

    idx = _knn_topk(off_y, ysq[:, None], xsT_pad, xsq_pad,
                    lo_arr, nt_arr, block_rows, col_tile)

    oxp = jnp.pad(off_x, ((0, 0), (0, 13)))
    oyp = jnp.pad(off_y, ((0, 0), (0, 13)))
    xi = _interp_sc(x, oxp, oyp, (idx[0], idx[1], idx[2]))

    out = _linear(xi, x_skip, W, b)
    return (out, pos_skip, batch_skip)


# speedup vs baseline: 1.0271x; 1.0271x over previous
import functools

import jax
import jax.numpy as jnp
from jax import lax
from jax.experimental import pallas as pl
from jax.experimental.pallas import tpu as pltpu
from jax.experimental.pallas import tpu_sc as plsc

KNN = 3
BATCH_OFF = 1000.0
_BIG = 2**30


def _insert3(state, cand):
    out = []
    e = cand
    for s in state:
        t = e[0] < s[0]
        keep = tuple(jnp.where(t, a, b) for a, b in zip(e, s))
        e = tuple(jnp.where(t, b, a) for a, b in zip(e, s))
        out.append(keep)
    return tuple(out)


def _knn_body(lo_ref, nt_ref, yc_ref, ysq_ref, xs_ref, xsq_ref, idx_ref,
              *, block_rows, col_tile):
    blk = pl.program_id(0)
    lo = lo_ref[blk]
    ntiles = nt_ref[blk]
    yc = yc_ref[...]
    ysq = ysq_ref[...]

    r = block_rows
    inf = jnp.float32(jnp.inf)
    siota = jax.lax.broadcasted_iota(jnp.int32, (col_tile, r), 0)

    def tile_step(t, state):
        start = pl.multiple_of(lo + t * col_tile, 128)
        xsT = xs_ref[:, pl.ds(start, col_tile)]
        xsq = xsq_ref[:, pl.ds(start, col_tile)]
        dot = jax.lax.dot_general(yc, xsT, (((1,), (0,)), ((), ())),
                                  preferred_element_type=jnp.float32)
        d2 = jnp.transpose((ysq + xsq) - 2.0 * dot)
        for k in range(KNN):
            m = jnp.min(d2, axis=0, keepdims=True)
            cand = jnp.where(d2 == m, siota, jnp.int32(_BIG))
            il = jnp.min(cand, axis=0, keepdims=True)
            state = _insert3(state, (m, il + start))
            if k < KNN - 1:
                d2 = jnp.where(siota == il, inf, d2)
        return state

    def entry():
        return (jnp.full((1, r), inf), jnp.full((1, r), _BIG, jnp.int32))

    state = jax.lax.fori_loop(0, ntiles, tile_step,
                              (entry(), entry(), entry()))
    for k in range(KNN):
        idx_ref[k, :] = state[k][1][0]


def _knn_topk(off_y, ysq, xsT_pad, xsq_pad, lo_arr, nt_arr,
              block_rows=256, col_tile=512):
    ny = off_y.shape[0]
    nxp = xsT_pad.shape[1]
    grid = (ny // block_rows,)
    body = functools.partial(_knn_body, block_rows=block_rows,
                             col_tile=col_tile)
    return pl.pallas_call(
        body,
        grid_spec=pltpu.PrefetchScalarGridSpec(
            num_scalar_prefetch=2,
            grid=grid,
            in_specs=[
                pl.BlockSpec((block_rows, 3), lambda i, lo, nt: (i, 0)),
                pl.BlockSpec((block_rows, 1), lambda i, lo, nt: (i, 0)),
                pl.BlockSpec((3, nxp), lambda i, lo, nt: (0, 0)),
                pl.BlockSpec((1, nxp), lambda i, lo, nt: (0, 0)),
            ],
            out_specs=pl.BlockSpec((KNN, block_rows),
                                   lambda i, lo, nt: (0, i)),
        ),
        out_shape=jax.ShapeDtypeStruct((KNN, ny), jnp.int32),
    )(lo_arr, nt_arr, off_y, ysq, xsT_pad, xsq_pad)


def _lane_bcast(v, lane):
    idx = jnp.full((16, 1), lane, jnp.int32)
    dn = lax.GatherDimensionNumbers(offset_dims=(), collapsed_slice_dims=(0,),
                                    start_index_map=(0,))
    return lax.gather(v, idx, dn, (1,),
                      mode=lax.GatherScatterMode.PROMISE_IN_BOUNDS)


def _interp_sc(x, oxp, oyp, idx_rows, chunk=128):
    nx, f = x.shape
    ny = idx_rows[0].shape[0]
    info = plsc.get_sparse_core_info()
    nw = info.num_cores * info.num_subcores
    rw = ny // nw
    nchunks = rw // chunk
    mesh = plsc.VectorSubcoreMesh(core_axis_name="c", subcore_axis_name="s")

    def body(x_h, oxp_h, oyp_h, i0_h, i1_h, i2_h, out_h,
             iv0, iv1, iv2, p0, p1, p2, yv, g0, g1, g2, ov, sem):
        wid = lax.axis_index("s") * info.num_cores + lax.axis_index("c")

        def chunk_body(ci, carry):
            base = wid * rw + ci * chunk
            pltpu.sync_copy(i0_h.at[pl.ds(base, chunk)], iv0)
            pltpu.sync_copy(i1_h.at[pl.ds(base, chunk)], iv1)
            pltpu.sync_copy(i2_h.at[pl.ds(base, chunk)], iv2)
            c0 = pltpu.async_copy(x_h.at[iv0], g0, sem)
            c1 = pltpu.async_copy(x_h.at[iv1], g1, sem)
            c2 = pltpu.async_copy(x_h.at[iv2], g2, sem)
            c3 = pltpu.async_copy(oxp_h.at[iv0], p0, sem)
            c4 = pltpu.async_copy(oxp_h.at[iv1], p1, sem)
            c5 = pltpu.async_copy(oxp_h.at[iv2], p2, sem)
            pltpu.sync_copy(oyp_h.at[pl.ds(base, chunk)], yv)
            for c in (c0, c1, c2, c3, c4, c5):
                c.wait()

            def one_row(r):
                yrow = yv[r, :]

                def d2sel(pref):
                    dd = yrow - pref[r, :]
                    s = dd * dd
                    d2s = (_lane_bcast(s, 0) + _lane_bcast(s, 1)) + _lane_bcast(s, 2)
                    return jnp.maximum(d2s, 1e-16)

                d0 = d2sel(p0)
                d1 = d2sel(p1)
                d2_ = d2sel(p2)
                p01 = d0 * d1
                p02 = d0 * d2_
                p12 = d1 * d2_
                inv = 1.0 / ((p12 + p02) + p01)
                a0 = p12 * inv
                a1 = p02 * inv
                a2 = p01 * inv
                for fi in range(f // 16):
                    fs = pl.ds(fi * 16, 16)
                    ov[r, fs] = (a0 * g0[r, fs] + a1 * g1[r, fs]) + a2 * g2[r, fs]

            unroll = 4

            def row_body(ri, c):
                for u in range(unroll):
                    one_row(ri * unroll + u)
                return c

            lax.fori_loop(0, chunk // unroll, row_body, 0)
            pltpu.sync_copy(ov, out_h.at[pl.ds(base, chunk)])
            return carry

        lax.fori_loop(0, nchunks, chunk_body, 0)

    return pl.kernel(
        body,
        out_type=jax.ShapeDtypeStruct((ny, f), jnp.float32),
        mesh=mesh,
        compiler_params=pltpu.CompilerParams(use_tc_tiling_on_sc=False),
        scratch_types=[
            pltpu.VMEM((chunk,), jnp.int32),
            pltpu.VMEM((chunk,), jnp.int32),
            pltpu.VMEM((chunk,), jnp.int32),
            pltpu.VMEM((chunk, 16), jnp.float32),
            pltpu.VMEM((chunk, 16), jnp.float32),
            pltpu.VMEM((chunk, 16), jnp.float32),
            pltpu.VMEM((chunk, 16), jnp.float32),
            pltpu.VMEM((chunk, f), jnp.float32),
            pltpu.VMEM((chunk, f), jnp.float32),
            pltpu.VMEM((chunk, f), jnp.float32),
            pltpu.VMEM((chunk, f), jnp.float32),
            pltpu.SemaphoreType.DMA,
        ],
    )(x, oxp, oyp, idx_rows[0], idx_rows[1], idx_rows[2])


def _lin_body(xi_ref, xs_ref, w1_ref, w2_ref, b_ref, o_ref):
    acc = jax.lax.dot_general(xi_ref[...], w1_ref[...],
                              (((1,), (0,)), ((), ())),
                              preferred_element_type=jnp.float32)
    acc = acc + jax.lax.dot_general(xs_ref[...], w2_ref[...],
                                    (((1,), (0,)), ((), ())),
                                    preferred_element_type=jnp.float32)
    o_ref[...] = acc + b_ref[...]


def _linear(xi, x_skip, W, b, block_rows=1024):
    ny, f = xi.shape
    w1 = W[:f]
    w2 = W[f:]
    b2d = b[None, :]
    grid = (ny // block_rows,)
    return pl.pallas_call(
        _lin_body,
        grid=grid,
        in_specs=[
            pl.BlockSpec((block_rows, f), lambda i: (i, 0)),
            pl.BlockSpec((block_rows, f), lambda i: (i, 0)),
            pl.BlockSpec((f, f), lambda i: (0, 0)),
            pl.BlockSpec((f, f), lambda i: (0, 0)),
            pl.BlockSpec((1, f), lambda i: (0, 0)),
        ],
        out_specs=pl.BlockSpec((block_rows, f), lambda i: (i, 0)),
        out_shape=jax.ShapeDtypeStruct((ny, f), jnp.float32),
    )(xi, x_skip, w1, w2, b2d)


def kernel(x, pos, batch, x_skip, pos_skip, batch_skip, W, b):
    nx = x.shape[0]
    ny = x_skip.shape[0]
    block_rows = 256
    col_tile = 512

    off_x = pos + BATCH_OFF * batch[:, None].astype(pos.dtype)
    off_y = pos_skip + BATCH_OFF * batch_skip[:, None].astype(pos_skip.dtype)
    xsq = jnp.sum(off_x * off_x, axis=-1)
    ysq = jnp.sum(off_y * off_y, axis=-1)

    nblk = ny // block_rows
    bs2 = batch_skip.reshape(nblk, block_rows)
    bmin = bs2[:, 0]
    bmax = bs2[:, -1]
    vals = jnp.arange(8, dtype=batch.dtype)
    starts = jnp.sum(batch[None, :] < vals[:, None], axis=1).astype(jnp.int32)
    ends = jnp.sum(batch[None, :] <= vals[:, None], axis=1).astype(jnp.int32)
    seg_lo = starts[bmin]
    seg_hi = ends[bmax]
    lo_arr = (seg_lo // 128) * 128
    nt_arr = (seg_hi - lo_arr + (col_tile - 1)) // col_tile

    xsT_pad = jnp.pad(off_x.T, ((0, 0), (0, col_tile)))
    xsq_pad = jnp.pad(xsq[None, :], ((0, 0), (0, col_tile)))

    oxp = jnp.pad(off_x, ((0, 0), (0, 13)))
    oyp = jnp.pad(off_y, ((0, 0), (0, 13)))

    h = ny // 2
    hb = h // block_rows
    ysq2 = ysq[:, None]
    xis = []
    idx_halves = []
    for a in range(2):
        rows = slice(a * h, (a + 1) * h)
        blks = slice(a * hb, (a + 1) * hb)
        idx_halves.append(_knn_topk(off_y[rows], ysq2[rows], xsT_pad,
                                    xsq_pad, lo_arr[blks], nt_arr[blks],
                                    block_rows, col_tile))
    for a in range(2):
        idx = idx_halves[a]
        rows = slice(a * h, (a + 1) * h)
        xis.append(_interp_sc(x, oxp, oyp[rows],
                              (idx[0], idx[1], idx[2])))

    xi = jnp.concatenate(xis, axis=0)
    out = _linear(xi, x_skip, W, b)
    return (out, pos_skip, batch_skip)


# speedup vs baseline: 1.0668x; 1.0387x over previous
import functools

import jax
import jax.numpy as jnp
from jax import lax
from jax.experimental import pallas as pl
from jax.experimental.pallas import tpu as pltpu
from jax.experimental.pallas import tpu_sc as plsc

KNN = 3
BATCH_OFF = 1000.0
_BIG = 2**30


def _insert3(state, cand):
    out = []
    e = cand
    for s in state:
        t = e[0] < s[0]
        keep = tuple(jnp.where(t, a, b) for a, b in zip(e, s))
        e = tuple(jnp.where(t, b, a) for a, b in zip(e, s))
        out.append(keep)
    return tuple(out)


def _knn_body(lo_ref, nt_ref, yc_ref, ysq_ref, xs_ref, xsq_ref, idx_ref,
              *, block_rows, col_tile):
    blk = pl.program_id(0)
    lo = lo_ref[blk]
    ntiles = nt_ref[blk]
    yc = yc_ref[...]
    ysq = ysq_ref[...]

    r = block_rows
    inf = jnp.float32(jnp.inf)
    siota = jax.lax.broadcasted_iota(jnp.int32, (col_tile, r), 0)

    def tile_step(t, state):
        start = pl.multiple_of(lo + t * col_tile, 128)
        xsT = xs_ref[:, pl.ds(start, col_tile)]
        xsq = xsq_ref[:, pl.ds(start, col_tile)]
        dot = jax.lax.dot_general(yc, xsT, (((1,), (0,)), ((), ())),
                                  preferred_element_type=jnp.float32)
        d2 = jnp.transpose((ysq + xsq) - 2.0 * dot)
        for k in range(KNN):
            m = jnp.min(d2, axis=0, keepdims=True)
            cand = jnp.where(d2 == m, siota, jnp.int32(_BIG))
            il = jnp.min(cand, axis=0, keepdims=True)
            state = _insert3(state, (m, il + start))
            if k < KNN - 1:
                d2 = jnp.where(siota == il, inf, d2)
        return state

    def entry():
        return (jnp.full((1, r), inf), jnp.full((1, r), _BIG, jnp.int32))

    state = jax.lax.fori_loop(0, ntiles, tile_step,
                              (entry(), entry(), entry()))
    for k in range(KNN):
        idx_ref[k, :] = state[k][1][0]


def _knn_topk(off_y, ysq, xsT_pad, xsq_pad, lo_arr, nt_arr,
              block_rows=256, col_tile=512):
    ny = off_y.shape[0]
    nxp = xsT_pad.shape[1]
    grid = (ny // block_rows,)
    body = functools.partial(_knn_body, block_rows=block_rows,
                             col_tile=col_tile)
    return pl.pallas_call(
        body,
        grid_spec=pltpu.PrefetchScalarGridSpec(
            num_scalar_prefetch=2,
            grid=grid,
            in_specs=[
                pl.BlockSpec((block_rows, 3), lambda i, lo, nt: (i, 0)),
                pl.BlockSpec((block_rows, 1), lambda i, lo, nt: (i, 0)),
                pl.BlockSpec((3, nxp), lambda i, lo, nt: (0, 0)),
                pl.BlockSpec((1, nxp), lambda i, lo, nt: (0, 0)),
            ],
            out_specs=pl.BlockSpec((KNN, block_rows),
                                   lambda i, lo, nt: (0, i)),
        ),
        out_shape=jax.ShapeDtypeStruct((KNN, ny), jnp.int32),
    )(lo_arr, nt_arr, off_y, ysq, xsT_pad, xsq_pad)


def _lane_bcast(v, lane):
    idx = jnp.full((16, 1), lane, jnp.int32)
    dn = lax.GatherDimensionNumbers(offset_dims=(), collapsed_slice_dims=(0,),
                                    start_index_map=(0,))
    return lax.gather(v, idx, dn, (1,),
                      mode=lax.GatherScatterMode.PROMISE_IN_BOUNDS)


def _interp_sc(x, oxp, oyp, idx_rows, chunk=64):
    nx, f = x.shape
    ny = idx_rows[0].shape[0]
    info = plsc.get_sparse_core_info()
    nw = info.num_cores * info.num_subcores
    rw = ny // nw
    nchunks = rw // chunk
    mesh = plsc.VectorSubcoreMesh(core_axis_name="c", subcore_axis_name="s")

    def body(x_h, oxp_h, oyp_h, i0_h, i1_h, i2_h, out_h,
             iva, yva, g, p, ov, sem_g, sem_out):
        wid = lax.axis_index("s") * info.num_cores + lax.axis_index("c")
        wbase = wid * rw
        pltpu.sync_copy(i0_h.at[pl.ds(wbase, rw)], iva.at[0])
        pltpu.sync_copy(i1_h.at[pl.ds(wbase, rw)], iva.at[1])
        pltpu.sync_copy(i2_h.at[pl.ds(wbase, rw)], iva.at[2])
        pltpu.sync_copy(oyp_h.at[pl.ds(wbase, rw)], yva)

        def gathers(ci, par):
            cs = [None] * 6
            for k in range(KNN):
                isl = iva.at[k, pl.ds(ci * chunk, chunk)]
                cs[k] = pltpu.make_async_copy(x_h.at[isl], g.at[par, k],
                                              sem_g[par])
                cs[3 + k] = pltpu.make_async_copy(oxp_h.at[isl], p.at[par, k],
                                                  sem_g[par])
            return cs

        def out_copy(ci, par):
            base = wbase + ci * chunk
            return pltpu.make_async_copy(ov.at[par],
                                         out_h.at[pl.ds(base, chunk)],
                                         sem_out[par])

        for c in gathers(0, 0):
            c.start()

        def compute(ci, par):
            def one_row(r):
                yrow = yva[ci * chunk + r, :]

                def d2sel(k):
                    dd = yrow - p[par, k, r, :]
                    s = dd * dd
                    d2s = (_lane_bcast(s, 0) + _lane_bcast(s, 1)) + _lane_bcast(s, 2)
                    return jnp.maximum(d2s, 1e-16)

                d0 = d2sel(0)
                d1 = d2sel(1)
                d2_ = d2sel(2)
                p01 = d0 * d1
                p02 = d0 * d2_
                p12 = d1 * d2_
                inv = 1.0 / ((p12 + p02) + p01)
                a0 = p12 * inv
                a1 = p02 * inv
                a2 = p01 * inv
                for fi in range(f // 16):
                    fs = pl.ds(fi * 16, 16)
                    ov[par, r, fs] = ((a0 * g[par, 0, r, fs]
                                       + a1 * g[par, 1, r, fs])
                                      + a2 * g[par, 2, r, fs])

            unroll = 4

            def row_body(ri, c):
                for u in range(unroll):
                    one_row(ri * unroll + u)
                return c

            lax.fori_loop(0, chunk // unroll, row_body, 0)

        ncj = nchunks // 2

        def pair_body(cj, carry):
            for par in (0, 1):
                ci = cj * 2 + par
                nxt = 1 - par
                for c in gathers(ci, par):
                    c.wait()

                if par == 0:
                    for c in gathers(ci + 1, nxt):
                        c.start()
                else:
                    @pl.when(cj < ncj - 1)
                    def _():
                        for c in gathers(ci + 1, nxt):
                            c.start()

                @pl.when(cj > 0)
                def _():
                    out_copy(ci - 2, par).wait()

                compute(ci, par)
                out_copy(ci, par).start()
            return carry

        lax.fori_loop(0, ncj, pair_body, 0)
        out_copy(nchunks - 2, 0).wait()
        out_copy(nchunks - 1, 1).wait()

    return pl.kernel(
        body,
        out_type=jax.ShapeDtypeStruct((ny, f), jnp.float32),
        mesh=mesh,
        compiler_params=pltpu.CompilerParams(use_tc_tiling_on_sc=False),
        scratch_types=[
            pltpu.VMEM((KNN, rw), jnp.int32),
            pltpu.VMEM((rw, 16), jnp.float32),
            pltpu.VMEM((2, KNN, chunk, f), jnp.float32),
            pltpu.VMEM((2, KNN, chunk, 16), jnp.float32),
            pltpu.VMEM((2, chunk, f), jnp.float32),
            [pltpu.SemaphoreType.DMA, pltpu.SemaphoreType.DMA],
            [pltpu.SemaphoreType.DMA, pltpu.SemaphoreType.DMA],
        ],
    )(x, oxp, oyp, idx_rows[0], idx_rows[1], idx_rows[2])


def _lin_body(xi_ref, xs_ref, w1_ref, w2_ref, b_ref, o_ref):
    acc = jax.lax.dot_general(xi_ref[...], w1_ref[...],
                              (((1,), (0,)), ((), ())),
                              preferred_element_type=jnp.float32)
    acc = acc + jax.lax.dot_general(xs_ref[...], w2_ref[...],
                                    (((1,), (0,)), ((), ())),
                                    preferred_element_type=jnp.float32)
    o_ref[...] = acc + b_ref[...]


def _linear(xi, x_skip, W, b, block_rows=1024):
    ny, f = xi.shape
    w1 = W[:f]
    w2 = W[f:]
    b2d = b[None, :]
    grid = (ny // block_rows,)
    return pl.pallas_call(
        _lin_body,
        grid=grid,
        in_specs=[
            pl.BlockSpec((block_rows, f), lambda i: (i, 0)),
            pl.BlockSpec((block_rows, f), lambda i: (i, 0)),
            pl.BlockSpec((f, f), lambda i: (0, 0)),
            pl.BlockSpec((f, f), lambda i: (0, 0)),
            pl.BlockSpec((1, f), lambda i: (0, 0)),
        ],
        out_specs=pl.BlockSpec((block_rows, f), lambda i: (i, 0)),
        out_shape=jax.ShapeDtypeStruct((ny, f), jnp.float32),
    )(xi, x_skip, w1, w2, b2d)


def kernel(x, pos, batch, x_skip, pos_skip, batch_skip, W, b):
    nx = x.shape[0]
    ny = x_skip.shape[0]
    block_rows = 256
    col_tile = 512

    off_x = pos + BATCH_OFF * batch[:, None].astype(pos.dtype)
    off_y = pos_skip + BATCH_OFF * batch_skip[:, None].astype(pos_skip.dtype)
    xsq = jnp.sum(off_x * off_x, axis=-1)
    ysq = jnp.sum(off_y * off_y, axis=-1)

    nblk = ny // block_rows
    bs2 = batch_skip.reshape(nblk, block_rows)
    bmin = bs2[:, 0]
    bmax = bs2[:, -1]
    vals = jnp.arange(8, dtype=batch.dtype)
    starts = jnp.sum(batch[None, :] < vals[:, None], axis=1).astype(jnp.int32)
    ends = jnp.sum(batch[None, :] <= vals[:, None], axis=1).astype(jnp.int32)
    seg_lo = starts[bmin]
    seg_hi = ends[bmax]
    lo_arr = (seg_lo // 128) * 128
    nt_arr = (seg_hi - lo_arr + (col_tile - 1)) // col_tile

    xsT_pad = jnp.pad(off_x.T, ((0, 0), (0, col_tile)))
    xsq_pad = jnp.pad(xsq[None, :], ((0, 0), (0, col_tile)))

    oxp = jnp.pad(off_x, ((0, 0), (0, 13)))
    oyp = jnp.pad(off_y, ((0, 0), (0, 13)))

    h = ny // 2
    hb = h // block_rows
    ysq2 = ysq[:, None]
    xis = []
    idx_halves = []
    for a in range(2):
        rows = slice(a * h, (a + 1) * h)
        blks = slice(a * hb, (a + 1) * hb)
        idx_halves.append(_knn_topk(off_y[rows], ysq2[rows], xsT_pad,
                                    xsq_pad, lo_arr[blks], nt_arr[blks],
                                    block_rows, col_tile))
    for a in range(2):
        idx = idx_halves[a]
        rows = slice(a * h, (a + 1) * h)
        xis.append(_interp_sc(x, oxp, oyp[rows],
                              (idx[0], idx[1], idx[2])))

    xi = jnp.concatenate(xis, axis=0)
    out = _linear(xi, x_skip, W, b)
    return (out, pos_skip, batch_skip)
